# baseline (speedup 1.0000x reference)
import functools

import jax
import jax.numpy as jnp
from jax import lax
from jax.experimental import pallas as pl
from jax.experimental.pallas import tpu as pltpu
from jax.experimental.pallas import tpu_sc as plsc

BAG = 16384
SC_N = 9728
TC_N = BAG - SC_N
EMBED_DIM = 64
HIDDEN = 128
NC, NS, L = 2, 16, 16
NW = NC * NS
B_PER_W = SC_N // NW
NB = 8
PANEL = 128
NVEC = EMBED_DIM // L
TNB = 256

_mesh = plsc.VectorSubcoreMesh(core_axis_name="c", subcore_axis_name="s")


@functools.partial(
    pl.kernel,
    mesh=_mesh,
    out_type=jax.ShapeDtypeStruct((NW * EMBED_DIM,), jnp.float32),
    scratch_types=[
        pltpu.VMEM((B_PER_W,), jnp.int32),
        pltpu.VMEM((NB, EMBED_DIM, PANEL), jnp.float32),
        pltpu.VMEM((EMBED_DIM,), jnp.float32),
    ] + [pltpu.SemaphoreType.DMA] * NB,
    compiler_params=pltpu.CompilerParams(use_tc_tiling_on_sc=True,
                                         needs_layout_passes=False),
)
def _sc_gather_sum(idx_hbm, tabt_hbm, out_hbm, idx_v, ring_v, acc_v, *sems):
    wid = lax.axis_index("s") * NC + lax.axis_index("c")
    pltpu.sync_copy(idx_hbm.at[pl.ds(wid * B_PER_W, B_PER_W)], idx_v)

    lane = jax.lax.iota(jnp.int32, L)

    def issue(idx, slot):
        tb = pl.multiple_of(jnp.bitwise_and(idx, -PANEL), PANEL)
        pltpu.async_copy(
            tabt_hbm.at[:, pl.ds(tb, PANEL)], ring_v.at[slot], sems[slot]
        )

    def wait_acc(idx, slot, accs):
        pltpu.make_async_copy(
            tabt_hbm.at[:, pl.ds(0, PANEL)], ring_v.at[slot], sems[slot]
        ).wait()
        off = jnp.full((L,), jnp.bitwise_and(idx, PANEL - 1), jnp.int32)
        slot_vec = jnp.full((L,), slot, jnp.int32)
        return tuple(
            accs[k] + plsc.load_gather(ring_v, [slot_vec, lane + k * L, off])
            for k in range(NVEC)
        )

    chunk0 = idx_v[pl.ds(0, L)]
    for s in range(NB):
        issue(chunk0[s], s)

    def body(g, carry):
        accs, cur = carry[:-1], carry[-1]
        nxt = idx_v[pl.ds((g + 1) * L, L)]
        for s in range(L):
            accs = wait_acc(cur[s], s % NB, accs)
            issue(cur[s + NB] if s < L - NB else nxt[s - (L - NB)], s % NB)
        return (*accs, nxt)

    zero = jnp.zeros((L,), jnp.float32)
    carry = lax.fori_loop(0, B_PER_W // L - 1, body,
                          (*(zero,) * NVEC, chunk0))
    accs, cur = carry[:-1], carry[-1]
    for s in range(L):
        accs = wait_acc(cur[s], s % NB, accs)
        if s < L - NB:
            issue(cur[s + NB], s % NB)

    for k in range(NVEC):
        acc_v[pl.ds(k * L, L)] = accs[k]
    pltpu.sync_copy(acc_v, out_hbm.at[pl.ds(wid * EMBED_DIM, EMBED_DIM)])


def _tc_gather(idx_ref, tabt_ref, out_ref, ring, sems):
    lane2d = lax.broadcasted_iota(jnp.int32, (EMBED_DIM, PANEL), 1)

    def issue(j, slot):
        c = idx_ref[j]
        tb = pl.multiple_of(jnp.bitwise_and(c, -PANEL), PANEL)
        pltpu.make_async_copy(
            tabt_ref.at[:, pl.ds(tb, PANEL)], ring.at[slot], sems.at[slot]
        ).start()

    for s in range(TNB):
        issue(s, s)

    def group(base, acc, refill):
        for s in range(TNB):
            pltpu.make_async_copy(
                tabt_ref.at[:, pl.ds(0, PANEL)], ring.at[s], sems.at[s]
            ).wait()
            c = idx_ref[base + s]
            cm = jnp.bitwise_and(c, PANEL - 1)
            acc = acc + jnp.where(lane2d == cm, ring[s], 0.0)
            if refill:
                issue(base + s + TNB, s)
        return acc

    acc = lax.fori_loop(
        0, TC_N // TNB - 1,
        lambda g, a: group(g * TNB, a, True),
        jnp.zeros((EMBED_DIM, PANEL), jnp.float32))
    acc = group(TC_N - TNB, acc, False)
    out_ref[...] = acc


_tc_gather_call = pl.pallas_call(
    _tc_gather,
    in_specs=[
        pl.BlockSpec(memory_space=pltpu.SMEM),
        pl.BlockSpec(memory_space=pltpu.HBM),
    ],
    out_specs=pl.BlockSpec(memory_space=pltpu.VMEM),
    out_shape=jax.ShapeDtypeStruct((EMBED_DIM, PANEL), jnp.float32),
    scratch_shapes=[
        pltpu.VMEM((TNB, EMBED_DIM, PANEL), jnp.float32),
        pltpu.SemaphoreType.DMA((TNB,)),
    ],
)


def _tc_head(partials_ref, tcacc_ref, hidden_ref, w1t_ref, b1_ref,
             w2t_ref, b2_ref, out_ref, hid_ref):
    sc_sum = jnp.sum(partials_ref[...], axis=0, keepdims=True)
    ones = jnp.ones((PANEL, 1), jnp.float32)
    col = jnp.dot(tcacc_ref[...], ones,
                  preferred_element_type=jnp.float32)
    eye = (lax.broadcasted_iota(jnp.int32, (EMBED_DIM, EMBED_DIM), 0)
           == lax.broadcasted_iota(jnp.int32, (EMBED_DIM, EMBED_DIM), 1)
           ).astype(jnp.float32)
    tc_sum = lax.dot_general(col, eye, (((0,), (0,)), ((), ())),
                             preferred_element_type=jnp.float32)
    emb = (sc_sum + tc_sum) * (1.0 / BAG)
    combined = jnp.concatenate([emb, hidden_ref[...]], axis=1)
    nh = jnp.dot(combined, w1t_ref[...],
                 preferred_element_type=jnp.float32) + b1_ref[...]
    hid_ref[...] = nh
    logit = jnp.dot(combined, w2t_ref[...],
                    preferred_element_type=jnp.float32) + b2_ref[...]
    out_ref[...] = 1.0 / (1.0 + jnp.exp(-logit))


_head = pl.pallas_call(
    _tc_head,
    out_shape=(
        jax.ShapeDtypeStruct((1, 1), jnp.float32),
        jax.ShapeDtypeStruct((1, HIDDEN), jnp.float32),
    ),
)


def kernel(input_, hidden, emb_table, W_i2h, b_i2h, W_i2o, b_i2o):
    tabt = emb_table.T
    partials = _sc_gather_sum(input_[:SC_N], tabt).reshape(NW, EMBED_DIM)
    tc_acc = _tc_gather_call(input_[SC_N:], tabt)
    output, new_hidden = _head(
        partials, tc_acc, hidden,
        W_i2h.T, b_i2h.reshape(1, HIDDEN),
        W_i2o.T, b_i2o.reshape(1, 1),
    )
    return (output, new_hidden)


# speedup vs baseline: 1.0264x; 1.0264x over previous
import functools

import jax
import jax.numpy as jnp
from jax import lax
from jax.experimental import pallas as pl
from jax.experimental.pallas import tpu as pltpu
from jax.experimental.pallas import tpu_sc as plsc

BAG = 16384
SC_N = 7168
TC_N = BAG - SC_N
EMBED_DIM = 64
HIDDEN = 128
NC, NS, L = 2, 16, 16
NW = NC * NS
B_PER_W = SC_N // NW
NB = 8
PANEL = 128
NVEC = EMBED_DIM // L
TNB = 256

_mesh = plsc.VectorSubcoreMesh(core_axis_name="c", subcore_axis_name="s")


@functools.partial(
    pl.kernel,
    mesh=_mesh,
    out_type=jax.ShapeDtypeStruct((NW * EMBED_DIM,), jnp.float32),
    scratch_types=[
        pltpu.VMEM((B_PER_W,), jnp.int32),
        pltpu.VMEM((NB, EMBED_DIM, PANEL), jnp.float32),
        pltpu.VMEM((EMBED_DIM,), jnp.float32),
    ] + [pltpu.SemaphoreType.DMA] * NB,
    compiler_params=pltpu.CompilerParams(use_tc_tiling_on_sc=True,
                                         needs_layout_passes=False),
)
def _sc_gather_sum(idx_hbm, tabt_hbm, out_hbm, idx_v, ring_v, acc_v, *sems):
    wid = lax.axis_index("s") * NC + lax.axis_index("c")
    pltpu.sync_copy(idx_hbm.at[pl.ds(wid * B_PER_W, B_PER_W)], idx_v)

    lane = jax.lax.iota(jnp.int32, L)

    def issue(idx, slot):
        tb = pl.multiple_of(jnp.bitwise_and(idx, -PANEL), PANEL)
        pltpu.async_copy(
            tabt_hbm.at[:, pl.ds(tb, PANEL)], ring_v.at[slot], sems[slot]
        )

    def wait_acc(idx, slot, accs):
        pltpu.make_async_copy(
            tabt_hbm.at[:, pl.ds(0, PANEL)], ring_v.at[slot], sems[slot]
        ).wait()
        off = jnp.full((L,), jnp.bitwise_and(idx, PANEL - 1), jnp.int32)
        slot_vec = jnp.full((L,), slot, jnp.int32)
        return tuple(
            accs[k] + plsc.load_gather(ring_v, [slot_vec, lane + k * L, off])
            for k in range(NVEC)
        )

    chunk0 = idx_v[pl.ds(0, L)]
    for s in range(NB):
        issue(chunk0[s], s)

    def body(g, carry):
        accs, cur = carry[:-1], carry[-1]
        nxt = idx_v[pl.ds((g + 1) * L, L)]
        for s in range(L):
            accs = wait_acc(cur[s], s % NB, accs)
            issue(cur[s + NB] if s < L - NB else nxt[s - (L - NB)], s % NB)
        return (*accs, nxt)

    zero = jnp.zeros((L,), jnp.float32)
    carry = lax.fori_loop(0, B_PER_W // L - 1, body,
                          (*(zero,) * NVEC, chunk0))
    accs, cur = carry[:-1], carry[-1]
    for s in range(L):
        accs = wait_acc(cur[s], s % NB, accs)
        if s < L - NB:
            issue(cur[s + NB], s % NB)

    for k in range(NVEC):
        acc_v[pl.ds(k * L, L)] = accs[k]
    pltpu.sync_copy(acc_v, out_hbm.at[pl.ds(wid * EMBED_DIM, EMBED_DIM)])


def _tc_gather(idx_ref, tabt_ref, out_ref, ring, sems):
    lane2d = lax.broadcasted_iota(jnp.int32, (EMBED_DIM, PANEL), 1)

    def issue(j, slot):
        c = idx_ref[j]
        tb = pl.multiple_of(jnp.bitwise_and(c, -PANEL), PANEL)
        pltpu.make_async_copy(
            tabt_ref.at[:, pl.ds(tb, PANEL)], ring.at[slot], sems.at[slot]
        ).start()

    for s in range(TNB):
        issue(s, s)

    def group(base, acc, refill):
        for s in range(TNB):
            pltpu.make_async_copy(
                tabt_ref.at[:, pl.ds(0, PANEL)], ring.at[s], sems.at[s]
            ).wait()
            c = idx_ref[base + s]
            cm = jnp.bitwise_and(c, PANEL - 1)
            acc = acc + jnp.where(lane2d == cm, ring[s], 0.0)
            if refill:
                issue(base + s + TNB, s)
        return acc

    acc = lax.fori_loop(
        0, TC_N // TNB - 1,
        lambda g, a: group(g * TNB, a, True),
        jnp.zeros((EMBED_DIM, PANEL), jnp.float32))
    acc = group(TC_N - TNB, acc, False)
    out_ref[...] = acc


_tc_gather_call = pl.pallas_call(
    _tc_gather,
    in_specs=[
        pl.BlockSpec(memory_space=pltpu.SMEM),
        pl.BlockSpec(memory_space=pltpu.HBM),
    ],
    out_specs=pl.BlockSpec(memory_space=pltpu.VMEM),
    out_shape=jax.ShapeDtypeStruct((EMBED_DIM, PANEL), jnp.float32),
    scratch_shapes=[
        pltpu.VMEM((TNB, EMBED_DIM, PANEL), jnp.float32),
        pltpu.SemaphoreType.DMA((TNB,)),
    ],
)


def _tc_head(partials_ref, tcacc_ref, hidden_ref, w1t_ref, b1_ref,
             w2t_ref, b2_ref, out_ref, hid_ref):
    sc_sum = jnp.sum(partials_ref[...], axis=0, keepdims=True)
    ones = jnp.ones((PANEL, 1), jnp.float32)
    col = jnp.dot(tcacc_ref[...], ones,
                  preferred_element_type=jnp.float32)
    eye = (lax.broadcasted_iota(jnp.int32, (EMBED_DIM, EMBED_DIM), 0)
           == lax.broadcasted_iota(jnp.int32, (EMBED_DIM, EMBED_DIM), 1)
           ).astype(jnp.float32)
    tc_sum = lax.dot_general(col, eye, (((0,), (0,)), ((), ())),
                             preferred_element_type=jnp.float32)
    emb = (sc_sum + tc_sum) * (1.0 / BAG)
    combined = jnp.concatenate([emb, hidden_ref[...]], axis=1)
    nh = jnp.dot(combined, w1t_ref[...],
                 preferred_element_type=jnp.float32) + b1_ref[...]
    hid_ref[...] = nh
    logit = jnp.dot(combined, w2t_ref[...],
                    preferred_element_type=jnp.float32) + b2_ref[...]
    out_ref[...] = 1.0 / (1.0 + jnp.exp(-logit))


_head = pl.pallas_call(
    _tc_head,
    out_shape=(
        jax.ShapeDtypeStruct((1, 1), jnp.float32),
        jax.ShapeDtypeStruct((1, HIDDEN), jnp.float32),
    ),
)


def kernel(input_, hidden, emb_table, W_i2h, b_i2h, W_i2o, b_i2o):
    tabt = emb_table.T
    partials = _sc_gather_sum(input_[:SC_N], tabt).reshape(NW, EMBED_DIM)
    tc_acc = _tc_gather_call(input_[SC_N:], tabt)
    output, new_hidden = _head(
        partials, tc_acc, hidden,
        W_i2h.T, b_i2h.reshape(1, HIDDEN),
        W_i2o.T, b_i2o.reshape(1, 1),
    )
    return (output, new_hidden)
